# Initial kernel scaffold; baseline (speedup 1.0000x reference)
import jax
import jax.numpy as jnp
from jax import lax
from jax.experimental import pallas as pl
from jax.experimental.pallas import tpu as pltpu
from jax.experimental.pallas import tpu_sc as plsc

_NC = 2
_NS = 16
_NW = _NC * _NS
_L = 16
_T = 3000
_CPR = 16
_ROWS = 4 * 64 * 8
_RPW = _ROWS // _NW


def _sc_body(x_hbm, ri_hbm, out_hbm, ri_v, buf, outb, sem):
    wid = lax.axis_index("s") * _NC + lax.axis_index("c")
    pltpu.sync_copy(ri_hbm, ri_v)

    def row_body(i, carry):
        p = wid * _RPW + i
        b = (p // 8) % 64
        seg = (p // 512) * 8 + (p % 8)
        rows16 = plsc.load_gather(
            ri_v, [jnp.full((_L,), seg, jnp.int32), lax.iota(jnp.int32, _L)]
        )
        idx = rows16 + b * 128
        pltpu.async_copy(x_hbm.at[idx], buf, sem).wait()

        def chunk(o):
            acc = buf[0, pl.ds(o, _L)]
            for r in range(1, _CPR):
                acc = acc + buf[r, pl.ds(o, _L)]
            outb[pl.ds(o, _L)] = acc * (1.0 / _CPR)

        def chunk_body(j, c):
            chunk(j * _L)
            return c

        lax.fori_loop(0, _T // _L, chunk_body, 0)
        chunk(_T - _L)
        pltpu.sync_copy(outb, out_hbm.at[p])
        return carry

    lax.fori_loop(0, _RPW, row_body, 0)


def kernel(x, region_indices):
    xf = x.reshape(64 * 128, _T)
    rif = region_indices.reshape(4 * 8, _CPR)
    mesh = plsc.VectorSubcoreMesh(core_axis_name="c", subcore_axis_name="s")
    out = pl.kernel(
        _sc_body,
        out_type=jax.ShapeDtypeStruct((_ROWS, _T), jnp.float32),
        mesh=mesh,
        scratch_types=[
            pltpu.VMEM((4 * 8, _CPR), jnp.int32),
            pltpu.VMEM((_CPR, _T), jnp.float32),
            pltpu.VMEM((_T,), jnp.float32),
            pltpu.SemaphoreType.DMA,
        ],
    )(xf, rif)
    return out.reshape(4, 64, 8, _T)


# speedup vs baseline: 1.0243x; 1.0243x over previous
import jax
import jax.numpy as jnp
from jax import lax
from jax.experimental import pallas as pl
from jax.experimental.pallas import tpu as pltpu
from jax.experimental.pallas import tpu_sc as plsc

_NC = 2
_NS = 16
_NW = _NC * _NS
_L = 16
_T = 3000
_CPR = 16
_ROWS = 4 * 64 * 8
_RPW = _ROWS // _NW


def _sc_body(x_hbm, ri_hbm, out_hbm, ri_v, buf, outb, sem):
    wid = lax.axis_index("s") * _NC + lax.axis_index("c")
    pltpu.sync_copy(ri_hbm, ri_v)

    def row_body(i, carry):
        p = wid * _RPW + i
        b = (p // 8) % 64
        seg = (p // 512) * 8 + (p % 8)
        idx = ri_v[seg, :] + b * 128
        pltpu.async_copy(x_hbm.at[idx], buf, sem).wait()

        def chunk(o):
            acc = buf[0, pl.ds(o, _L)]
            for r in range(1, _CPR):
                acc = acc + buf[r, pl.ds(o, _L)]
            outb[pl.ds(o, _L)] = acc * (1.0 / _CPR)

        def chunk_body(j, c):
            chunk(j * _L)
            return c

        lax.fori_loop(0, _T // _L, chunk_body, 0)
        chunk(_T - _L)
        pltpu.sync_copy(outb, out_hbm.at[p])
        return carry

    lax.fori_loop(0, _RPW, row_body, 0)


def kernel(x, region_indices):
    xf = x.reshape(64 * 128, _T)
    rif = region_indices.reshape(4 * 8, _CPR)
    mesh = plsc.VectorSubcoreMesh(core_axis_name="c", subcore_axis_name="s")
    out = pl.kernel(
        _sc_body,
        out_type=jax.ShapeDtypeStruct((_ROWS, _T), jnp.float32),
        mesh=mesh,
        scratch_types=[
            pltpu.VMEM((4 * 8, _CPR), jnp.int32),
            pltpu.VMEM((_CPR, _T), jnp.float32),
            pltpu.VMEM((_T,), jnp.float32),
            pltpu.SemaphoreType.DMA,
        ],
        compiler_params=pltpu.CompilerParams(use_tc_tiling_on_sc=False),
    )(xf, rif)
    return out.reshape(4, 64, 8, _T)


# speedup vs baseline: 1.3841x; 1.3513x over previous
import jax
import jax.numpy as jnp
from jax import lax
from jax.experimental import pallas as pl
from jax.experimental.pallas import tpu as pltpu
from jax.experimental.pallas import tpu_sc as plsc

_NC = 2
_NS = 16
_NW = _NC * _NS
_L = 16
_T = 3000
_CPR = 16
_ROWS = 4 * 64 * 8
_RPW = _ROWS // _NW


def _sc_body(x_hbm, ri_hbm, out_hbm, ri_v, buf, outb, gs0, gs1, os0, os1):
    wid = lax.axis_index("s") * _NC + lax.axis_index("c")
    pltpu.sync_copy(ri_hbm, ri_v)
    gsem = (gs0, gs1)
    osem = (os0, os1)

    def row_meta(i):
        p = wid * _RPW + i
        b = (p // 8) % 64
        seg = (p // 512) * 8 + (p % 8)
        return p, ri_v[seg, :] + b * 128

    def gather(i, k):
        p, idx = row_meta(i)
        return pltpu.make_async_copy(x_hbm.at[idx], buf.at[k], gsem[k])

    def out_copy(i, k):
        p, _ = row_meta(i)
        return pltpu.make_async_copy(outb.at[k], out_hbm.at[p], osem[k])

    def reduce_row(k):
        def chunk(o):
            acc = buf[k, 0, pl.ds(o, _L)]
            for r in range(1, _CPR):
                acc = acc + buf[k, r, pl.ds(o, _L)]
            outb[k, pl.ds(o, _L)] = acc * (1.0 / _CPR)

        def chunk_body(j, c):
            chunk(j * _L)
            return c

        lax.fori_loop(0, _T // _L, chunk_body, 0)
        chunk(_T - _L)

    gather(0, 0).start()
    gather(1, 1).start()

    def process(i, k, prefetch, wait_out):
        gather(i, k).wait()
        if wait_out:
            out_copy(i - 2, k).wait()
        reduce_row(k)
        if prefetch:
            gather(i + 2, k).start()
        out_copy(i, k).start()

    process(0, 0, True, False)
    process(1, 1, True, False)

    def step(g, c):
        process(2 * g + 0, 0, True, True)
        process(2 * g + 1, 1, True, True)
        return c

    lax.fori_loop(1, _RPW // 2 - 1, step, 0)

    process(_RPW - 2, 0, False, True)
    process(_RPW - 1, 1, False, True)
    out_copy(_RPW - 2, 0).wait()
    out_copy(_RPW - 1, 1).wait()


def kernel(x, region_indices):
    xf = x.reshape(64 * 128, _T)
    rif = region_indices.reshape(4 * 8, _CPR)
    mesh = plsc.VectorSubcoreMesh(core_axis_name="c", subcore_axis_name="s")
    out = pl.kernel(
        _sc_body,
        out_type=jax.ShapeDtypeStruct((_ROWS, _T), jnp.float32),
        mesh=mesh,
        scratch_types=[
            pltpu.VMEM((4 * 8, _CPR), jnp.int32),
            pltpu.VMEM((2, _CPR, _T), jnp.float32),
            pltpu.VMEM((2, _T), jnp.float32),
            pltpu.SemaphoreType.DMA,
            pltpu.SemaphoreType.DMA,
            pltpu.SemaphoreType.DMA,
            pltpu.SemaphoreType.DMA,
        ],
        compiler_params=pltpu.CompilerParams(use_tc_tiling_on_sc=False),
    )(xf, rif)
    return out.reshape(4, 64, 8, _T)
